# Initial kernel scaffold; baseline (speedup 1.0000x reference)
import jax
import jax.numpy as jnp
from jax.experimental import pallas as pl


def kernel(x, emb, W1, b1, g1, be1, W2, b2, g2, be2, W3, b3):
    raise NotImplementedError("write your pallas kernel here")


# speedup vs baseline: 2.7429x; 2.7429x over previous
import functools

import jax
import jax.numpy as jnp
from jax import lax
from jax.experimental import pallas as pl
from jax.experimental.pallas import tpu as pltpu
import jax.experimental.pallas.tpu_sc as plsc

B = 4096
NUM_WORDS = 200
VOCAB = 100000
EMB = 128
H1 = 64
H2 = 16
EPS = 1e-5

_NC = 2
_NS = 16
_NWORKERS = _NC * _NS

_CH = 128


def _sc_gather(emb, idx_flat):
    n = idx_flat.shape[0]
    per_w = n // _NWORKERS
    n_ch = per_w // _CH
    mesh = plsc.VectorSubcoreMesh(core_axis_name="c", subcore_axis_name="s")

    @functools.partial(
        pl.kernel,
        out_type=jax.ShapeDtypeStruct((n, EMB), jnp.float32),
        mesh=mesh,
        scratch_types=[
            pltpu.VMEM((_CH,), jnp.int32),
            pltpu.VMEM((_CH, EMB), jnp.float32),
            pltpu.SemaphoreType.DMA,
        ],
    )
    def k(emb_hbm, idx_hbm, out_hbm, idx_v, buf, sem):
        wid = lax.axis_index("s") * _NC + lax.axis_index("c")
        base = wid * per_w

        @pl.loop(0, n_ch)
        def _(c):
            off = base + c * _CH
            pltpu.sync_copy(idx_hbm.at[pl.ds(off, _CH)], idx_v)
            pltpu.async_copy(emb_hbm.at[idx_v], buf, sem).wait()
            pltpu.sync_copy(buf, out_hbm.at[pl.ds(off, _CH)])

    return k(emb, idx_flat)


_BT = 512
_KC = 3200
_NB = B // _BT
_NK = (NUM_WORDS * EMB) // _KC


def _mlp_body(e_ref, w1t_ref, b1_ref, g1_ref, be1_ref, w2t_ref, b2_ref,
              g2_ref, be2_ref, w3_ref, b3_ref, out_ref, acc_ref):
    kk = pl.program_id(0)
    bb = pl.program_id(1)
    part = jnp.dot(e_ref[...], w1t_ref[...], preferred_element_type=jnp.float32)
    sl = pl.ds(bb * _BT, _BT)

    @pl.when(kk == 0)
    def _():
        acc_ref[sl, :] = part

    @pl.when(kk > 0)
    def _():
        acc_ref[sl, :] = acc_ref[sl, :] + part

    @pl.when((kk == _NK - 1) & (bb == _NB - 1))
    def _():
        h1 = acc_ref[...] + b1_ref[...]
        mu1 = jnp.mean(h1, axis=0, keepdims=True)
        d1 = h1 - mu1
        var1 = jnp.mean(d1 * d1, axis=0, keepdims=True)
        r1 = jnp.maximum(d1 * (g1_ref[...] * lax.rsqrt(var1 + EPS)) + be1_ref[...], 0.0)
        h2 = jnp.dot(r1, w2t_ref[...], preferred_element_type=jnp.float32) + b2_ref[...]
        mu2 = jnp.mean(h2, axis=0, keepdims=True)
        d2 = h2 - mu2
        var2 = jnp.mean(d2 * d2, axis=0, keepdims=True)
        r2 = jnp.maximum(d2 * (g2_ref[...] * lax.rsqrt(var2 + EPS)) + be2_ref[...], 0.0)
        h3 = jnp.sum(r2 * w3_ref[...], axis=1, keepdims=True) + b3_ref[...]
        out_ref[...] = jax.nn.sigmoid(h3)


def _tc_mlp(e2d, w1t, b1, g1, be1, w2t, b2, g2, be2, w3, b3, interpret=False):
    smallspec = pl.BlockSpec((1, H1), lambda k, b: (0, 0))
    smallspec2 = pl.BlockSpec((1, H2), lambda k, b: (0, 0))
    return pl.pallas_call(
        _mlp_body,
        grid=(_NK, _NB),
        in_specs=[
            pl.BlockSpec((_BT, _KC), lambda k, b: (b, k)),
            pl.BlockSpec((_KC, H1), lambda k, b: (k, 0)),
            smallspec, smallspec, smallspec,
            pl.BlockSpec((H1, H2), lambda k, b: (0, 0)),
            smallspec2, smallspec2, smallspec2,
            smallspec2,
            pl.BlockSpec((1, 1), lambda k, b: (0, 0)),
        ],
        out_specs=pl.BlockSpec((B, 1), lambda k, b: (0, 0)),
        out_shape=jax.ShapeDtypeStruct((B, 1), jnp.float32),
        scratch_shapes=[pltpu.VMEM((B, H1), jnp.float32)],
        interpret=interpret,
    )(e2d, w1t, b1, g1, be1, w2t, b2, g2, be2, w3, b3)


def kernel(x, emb, W1, b1, g1, be1, W2, b2, g2, be2, W3, b3):
    idx_flat = x.reshape(-1)
    e = _sc_gather(emb, idx_flat)
    e2d = e.reshape(B, NUM_WORDS * EMB)
    return _tc_mlp(
        e2d, W1.T,
        b1.reshape(1, H1), g1.reshape(1, H1), be1.reshape(1, H1),
        W2.T,
        b2.reshape(1, H2), g2.reshape(1, H2), be2.reshape(1, H2),
        W3.reshape(1, H2), b3.reshape(1, 1),
    )


# speedup vs baseline: 3.6095x; 1.3160x over previous
import functools

import jax
import jax.numpy as jnp
from jax import lax
from jax.experimental import pallas as pl
from jax.experimental.pallas import tpu as pltpu
import jax.experimental.pallas.tpu_sc as plsc

B = 4096
NUM_WORDS = 200
VOCAB = 100000
EMB = 128
H1 = 64
H2 = 16
EPS = 1e-5

_NC = 2
_NS = 16
_NWORKERS = _NC * _NS

_CH = 128


_H = 2
_HR = _H * _CH


def _sc_gather(emb, idx_flat):
    n = idx_flat.shape[0]
    per_w = n // _NWORKERS
    n_half = per_w // _HR
    mesh = plsc.VectorSubcoreMesh(core_axis_name="c", subcore_axis_name="s")

    @functools.partial(
        pl.kernel,
        out_type=jax.ShapeDtypeStruct((n, EMB), jnp.float32),
        mesh=mesh,
        scratch_types=[
            pltpu.VMEM((per_w,), jnp.int32),
            pltpu.VMEM((_HR, EMB), jnp.float32),
            pltpu.VMEM((_HR, EMB), jnp.float32),
            pltpu.SemaphoreType.DMA,
            pltpu.SemaphoreType.DMA,
            pltpu.SemaphoreType.DMA,
            pltpu.SemaphoreType.DMA,
        ],
    )
    def k(emb_hbm, idx_hbm, out_hbm, idx_v, buf_a, buf_b, sga, sgb, swa, swb):
        wid = lax.axis_index("s") * _NC + lax.axis_index("c")
        base = wid * per_w
        bufs = (buf_a, buf_b)
        sg = (sga, sgb)
        sw = (swa, swb)
        pltpu.sync_copy(idx_hbm.at[pl.ds(base, per_w)], idx_v)

        def issue_gather(p, half):
            for j in range(_H):
                pltpu.async_copy(
                    emb_hbm.at[idx_v.at[pl.ds(half * _HR + j * _CH, _CH)]],
                    bufs[p].at[pl.ds(j * _CH, _CH)], sg[p])

        def wait_gather(p):
            for j in range(_H):
                pltpu.make_async_copy(
                    emb_hbm.at[idx_v.at[pl.ds(j * _CH, _CH)]],
                    bufs[p].at[pl.ds(j * _CH, _CH)], sg[p]).wait()

        def issue_wb(p, half):
            pltpu.async_copy(bufs[p], out_hbm.at[pl.ds(base + half * _HR, _HR)], sw[p])

        def wait_wb(p):
            pltpu.make_async_copy(bufs[p], out_hbm.at[pl.ds(base, _HR)], sw[p]).wait()

        def substep(h, p):
            q = 1 - p

            @pl.when(h >= 1)
            def _():
                wait_wb(q)

            @pl.when(h + 1 < n_half)
            def _():
                issue_gather(q, h + 1)

            wait_gather(p)
            issue_wb(p, h)

        issue_gather(0, 0)

        @pl.loop(0, n_half, step=2)
        def _(h):
            substep(h, 0)
            substep(h + 1, 1)

        wait_wb(1)

    return k(emb, idx_flat)


_BT = 512
_KC = 3200
_NB = B // _BT
_NK = (NUM_WORDS * EMB) // _KC


def _mlp_body(e_ref, w1t_ref, b1_ref, g1_ref, be1_ref, w2t_ref, b2_ref,
              g2_ref, be2_ref, w3_ref, b3_ref, out_ref, acc_ref):
    kk = pl.program_id(0)
    bb = pl.program_id(1)
    part = jnp.dot(e_ref[...], w1t_ref[...], preferred_element_type=jnp.float32)
    sl = pl.ds(bb * _BT, _BT)

    @pl.when(kk == 0)
    def _():
        acc_ref[sl, :] = part

    @pl.when(kk > 0)
    def _():
        acc_ref[sl, :] = acc_ref[sl, :] + part

    @pl.when((kk == _NK - 1) & (bb == _NB - 1))
    def _():
        h1 = acc_ref[...] + b1_ref[...]
        mu1 = jnp.mean(h1, axis=0, keepdims=True)
        d1 = h1 - mu1
        var1 = jnp.mean(d1 * d1, axis=0, keepdims=True)
        r1 = jnp.maximum(d1 * (g1_ref[...] * lax.rsqrt(var1 + EPS)) + be1_ref[...], 0.0)
        h2 = jnp.dot(r1, w2t_ref[...], preferred_element_type=jnp.float32) + b2_ref[...]
        mu2 = jnp.mean(h2, axis=0, keepdims=True)
        d2 = h2 - mu2
        var2 = jnp.mean(d2 * d2, axis=0, keepdims=True)
        r2 = jnp.maximum(d2 * (g2_ref[...] * lax.rsqrt(var2 + EPS)) + be2_ref[...], 0.0)
        h3 = jnp.sum(r2 * w3_ref[...], axis=1, keepdims=True) + b3_ref[...]
        out_ref[...] = jax.nn.sigmoid(h3)


def _tc_mlp(e2d, w1t, b1, g1, be1, w2t, b2, g2, be2, w3, b3, interpret=False):
    smallspec = pl.BlockSpec((1, H1), lambda k, b: (0, 0))
    smallspec2 = pl.BlockSpec((1, H2), lambda k, b: (0, 0))
    return pl.pallas_call(
        _mlp_body,
        grid=(_NK, _NB),
        in_specs=[
            pl.BlockSpec((_BT, _KC), lambda k, b: (b, k)),
            pl.BlockSpec((_KC, H1), lambda k, b: (k, 0)),
            smallspec, smallspec, smallspec,
            pl.BlockSpec((H1, H2), lambda k, b: (0, 0)),
            smallspec2, smallspec2, smallspec2,
            smallspec2,
            pl.BlockSpec((1, 1), lambda k, b: (0, 0)),
        ],
        out_specs=pl.BlockSpec((B, 1), lambda k, b: (0, 0)),
        out_shape=jax.ShapeDtypeStruct((B, 1), jnp.float32),
        scratch_shapes=[pltpu.VMEM((B, H1), jnp.float32)],
        interpret=interpret,
    )(e2d, w1t, b1, g1, be1, w2t, b2, g2, be2, w3, b3)


def kernel(x, emb, W1, b1, g1, be1, W2, b2, g2, be2, W3, b3):
    idx_flat = x.reshape(-1)
    e = _sc_gather(emb, idx_flat)
    e2d = e.reshape(B, NUM_WORDS * EMB)
    return _tc_mlp(
        e2d, W1.T,
        b1.reshape(1, H1), g1.reshape(1, H1), be1.reshape(1, H1),
        W2.T,
        b2.reshape(1, H2), g2.reshape(1, H2), be2.reshape(1, H2),
        W3.reshape(1, H2), b3.reshape(1, 1),
    )
